# Initial kernel scaffold; baseline (speedup 1.0000x reference)
import jax
import jax.numpy as jnp
from jax.experimental import pallas as pl


def kernel(x, edge_index, W_self1, W_neigh1, b1, W_self2, W_neigh2, b2, Wp, bp):
    raise NotImplementedError("write your pallas kernel here")


# speedup vs baseline: 5.1194x; 5.1194x over previous
import functools

import jax
import jax.numpy as jnp
from jax import lax
from jax.experimental import pallas as pl
from jax.experimental.pallas import tpu as pltpu
from jax.experimental.pallas import tpu_sc as plsc

_N = 10000
_E = 320000
_D = 128

_NC = 2
_NS = 16
_NW = _NC * _NS
_EPW = _E // _NW
_CH = 80
_NCHUNK = _EPW // _CH
_RPS = 624
_TAIL = _N - _RPS * _NS
_DEG_PAD = 10240
_DPS = _DEG_PAD // _NS


def _sc_agg_body(with_deg, h_hbm, src_hbm, dst_hbm, *refs):
    if with_deg:
        (out_hbm, deg_hbm, srcv, dstv, rowsv, onesv, acc, dacc, sem) = refs
    else:
        (out_hbm, srcv, dstv, rowsv, onesv, acc, sem) = refs
    c = lax.axis_index("c")
    s = lax.axis_index("s")
    w = s * _NC + c

    zero16 = jnp.zeros((16,), jnp.float32)
    def _zrow(i, carry):
        rowsv[i // (_D // 16), pl.ds((i % (_D // 16)) * 16, 16)] = zero16
        return carry
    lax.fori_loop(0, _CH * _D // 16, _zrow, 0)
    one16 = jnp.full((16,), 1.0, jnp.float32)
    for i in range(_CH // 16):
        onesv[pl.ds(i * 16, 16)] = one16

    for t in range(_RPS // _CH):
        pltpu.sync_copy(rowsv, acc.at[pl.ds(s * _RPS + t * _CH, _CH)])
    rem = _RPS % _CH
    if rem:
        pltpu.sync_copy(rowsv.at[pl.ds(0, rem)],
                        acc.at[pl.ds(s * _RPS + (_RPS // _CH) * _CH, rem)])

    @pl.when(s == _NS - 1)
    def _():
        pltpu.sync_copy(rowsv.at[pl.ds(0, _TAIL)],
                        acc.at[pl.ds(_RPS * _NS, _TAIL)])

    if with_deg:
        for t in range(_DPS // _D):
            pltpu.sync_copy(rowsv.at[0], dacc.at[pl.ds(s * _DPS + t * _D, _D)])
    plsc.subcore_barrier()

    base0 = w * _EPW
    def _step(j, carry):
        base = base0 + j * _CH
        pltpu.sync_copy(src_hbm.at[pl.ds(base, _CH)], srcv)
        pltpu.sync_copy(dst_hbm.at[pl.ds(base, _CH)], dstv)
        pltpu.async_copy(h_hbm.at[srcv], rowsv, sem).wait()
        pltpu.sync_copy(rowsv, acc.at[dstv], add=True)
        if with_deg:
            pltpu.sync_copy(onesv, dacc.at[dstv], add=True)
        return carry
    lax.fori_loop(0, _NCHUNK, _step, 0)
    plsc.subcore_barrier()

    pltpu.sync_copy(acc.at[pl.ds(s * _RPS, _RPS)],
                    out_hbm.at[c, pl.ds(s * _RPS, _RPS)])

    @pl.when(s == _NS - 1)
    def _():
        pltpu.sync_copy(acc.at[pl.ds(_RPS * _NS, _TAIL)],
                        out_hbm.at[c, pl.ds(_RPS * _NS, _TAIL)])

    if with_deg:
        pltpu.sync_copy(dacc.at[pl.ds(s * _DPS, _DPS)],
                        deg_hbm.at[c, pl.ds(s * _DPS, _DPS)])


def _make_sc_agg(with_deg):
    mesh = plsc.VectorSubcoreMesh(core_axis_name="c", subcore_axis_name="s")
    out_type = [jax.ShapeDtypeStruct((_NC, _N, _D), jnp.float32)]
    if with_deg:
        out_type.append(jax.ShapeDtypeStruct((_NC, _DEG_PAD), jnp.float32))
    scratch = [
        pltpu.VMEM((_CH,), jnp.int32),
        pltpu.VMEM((_CH,), jnp.int32),
        pltpu.VMEM((_CH, _D), jnp.float32),
        pltpu.VMEM((_CH,), jnp.float32),
        pltpu.VMEM_SHARED((_N, _D), jnp.float32),
    ]
    if with_deg:
        scratch.append(pltpu.VMEM_SHARED((_DEG_PAD,), jnp.float32))
    scratch.append(pltpu.SemaphoreType.DMA)
    return pl.kernel(
        functools.partial(_sc_agg_body, with_deg),
        out_type=tuple(out_type) if with_deg else out_type[0],
        mesh=mesh,
        scratch_types=scratch,
    )


_R = 2000


def _layer1_body(h_ref, agg_ref, deg_ref, ws_ref, wn_ref, b_ref, out_ref):
    a = agg_ref[0] + agg_ref[1]
    inv = 1.0 / jnp.maximum(deg_ref[...], 1.0)
    out_ref[...] = jnp.maximum(
        jnp.dot(h_ref[...], ws_ref[...], preferred_element_type=jnp.float32)
        + jnp.dot(a * inv, wn_ref[...], preferred_element_type=jnp.float32)
        + b_ref[...], 0.0)


def _layer2_body(h_ref, agg_ref, deg_ref, ws_ref, wn_ref, b_ref, wp_ref,
                 bp_ref, out_ref, colsum):
    i = pl.program_id(0)
    a = agg_ref[0] + agg_ref[1]
    inv = 1.0 / jnp.maximum(deg_ref[...], 1.0)
    h2 = jnp.maximum(
        jnp.dot(h_ref[...], ws_ref[...], preferred_element_type=jnp.float32)
        + jnp.dot(a * inv, wn_ref[...], preferred_element_type=jnp.float32)
        + b_ref[...], 0.0)

    @pl.when(i == 0)
    def _():
        colsum[...] = jnp.zeros_like(colsum)

    colsum[...] += jnp.sum(h2, axis=0, keepdims=True)

    @pl.when(i == pl.num_programs(0) - 1)
    def _():
        out_ref[...] = (
            jnp.dot(colsum[...] * (1.0 / _N), wp_ref[...],
                    preferred_element_type=jnp.float32) + bp_ref[...])


def _tc_layer1(h, agg, deg2d, Ws, Wn, b):
    return pl.pallas_call(
        _layer1_body,
        grid=(_N // _R,),
        in_specs=[
            pl.BlockSpec((_R, _D), lambda i: (i, 0)),
            pl.BlockSpec((_NC, _R, _D), lambda i: (0, i, 0)),
            pl.BlockSpec((_R, 1), lambda i: (i, 0)),
            pl.BlockSpec((_D, _D), lambda i: (0, 0)),
            pl.BlockSpec((_D, _D), lambda i: (0, 0)),
            pl.BlockSpec((1, _D), lambda i: (0, 0)),
        ],
        out_specs=pl.BlockSpec((_R, _D), lambda i: (i, 0)),
        out_shape=jax.ShapeDtypeStruct((_N, _D), jnp.float32),
    )(h, agg, deg2d, Ws, Wn, b)


def _tc_layer2(h, agg, deg2d, Ws, Wn, b, Wp, bp):
    return pl.pallas_call(
        _layer2_body,
        grid=(_N // _R,),
        in_specs=[
            pl.BlockSpec((_R, _D), lambda i: (i, 0)),
            pl.BlockSpec((_NC, _R, _D), lambda i: (0, i, 0)),
            pl.BlockSpec((_R, 1), lambda i: (i, 0)),
            pl.BlockSpec((_D, _D), lambda i: (0, 0)),
            pl.BlockSpec((_D, _D), lambda i: (0, 0)),
            pl.BlockSpec((1, _D), lambda i: (0, 0)),
            pl.BlockSpec((_D, 1), lambda i: (0, 0)),
            pl.BlockSpec((1, 1), lambda i: (0, 0)),
        ],
        out_specs=pl.BlockSpec((1, 1), lambda i: (0, 0)),
        out_shape=jax.ShapeDtypeStruct((1, 1), jnp.float32),
        scratch_shapes=[pltpu.VMEM((1, _D), jnp.float32)],
    )(h, agg, deg2d, Ws, Wn, b, Wp, bp)


def kernel(x, edge_index, W_self1, W_neigh1, b1, W_self2, W_neigh2, b2, Wp, bp):
    src = edge_index[0].astype(jnp.int32)
    dst = edge_index[1].astype(jnp.int32)

    agg1, degp = _make_sc_agg(True)(x, src, dst)
    deg2d = (degp[0, :_N] + degp[1, :_N]).reshape(_N, 1)
    h1 = _tc_layer1(x, agg1, deg2d, W_self1, W_neigh1, b1.reshape(1, _D))
    agg2 = _make_sc_agg(False)(h1, src, dst)
    return _tc_layer2(h1, agg2, deg2d, W_self2, W_neigh2, b2.reshape(1, _D),
                      Wp, bp.reshape(1, 1))


# speedup vs baseline: 9.1435x; 1.7861x over previous
import functools

import jax
import jax.numpy as jnp
from jax import lax
from jax.experimental import pallas as pl
from jax.experimental.pallas import tpu as pltpu
from jax.experimental.pallas import tpu_sc as plsc

_N = 10000
_E = 320000
_D = 128

_NC = 2
_NS = 16
_NW = _NC * _NS
_EPW = _E // _NW
_CH = 80
_NCHUNK = _EPW // _CH
_NSTG = 5
_CPS = _NCHUNK // _NSTG
_RPS = 624
_TAIL = _N - _RPS * _NS
_DEG_PAD = 10240
_DPS = _DEG_PAD // _NS


def _sc_agg_body(with_deg, h_hbm, src_hbm, dst_hbm, *refs):
    if with_deg:
        (out_hbm, deg_hbm, srcv, dstv, rows_a, rows_b, onesv, acc, dacc,
         sem_a, sem_b) = refs
    else:
        (out_hbm, srcv, dstv, rows_a, rows_b, onesv, acc, sem_a, sem_b) = refs
    c = lax.axis_index("c")
    s = lax.axis_index("s")
    w = s * _NC + c

    zero16 = jnp.zeros((16,), jnp.float32)
    def _zrow(i, carry):
        rows_a[i // (_D // 16), pl.ds((i % (_D // 16)) * 16, 16)] = zero16
        return carry
    lax.fori_loop(0, _CH * _D // 16, _zrow, 0)
    one16 = jnp.full((16,), 1.0, jnp.float32)
    for i in range(_CH // 16):
        onesv[pl.ds(i * 16, 16)] = one16

    for t in range(_RPS // _CH):
        pltpu.sync_copy(rows_a, acc.at[pl.ds(s * _RPS + t * _CH, _CH)])
    rem = _RPS % _CH
    if rem:
        pltpu.sync_copy(rows_a.at[pl.ds(0, rem)],
                        acc.at[pl.ds(s * _RPS + (_RPS // _CH) * _CH, rem)])

    @pl.when(s == _NS - 1)
    def _():
        pltpu.sync_copy(rows_a.at[pl.ds(0, _TAIL)],
                        acc.at[pl.ds(_RPS * _NS, _TAIL)])

    if with_deg:
        for t in range(_DPS // _D):
            pltpu.sync_copy(rows_a.at[0], dacc.at[pl.ds(s * _DPS + t * _D, _D)])
    plsc.subcore_barrier()

    def _gather(j, buf, sem):
        pltpu.async_copy(h_hbm.at[srcv.at[j]], buf, sem)

    def _gwait(j, buf, sem):
        pltpu.make_async_copy(h_hbm.at[srcv.at[j]], buf, sem).wait()

    def _scat(j, buf):
        pltpu.sync_copy(buf, acc.at[dstv.at[j]], add=True)
        if with_deg:
            pltpu.sync_copy(onesv, dacc.at[dstv.at[j]], add=True)

    def _stage(st, carry):
        pltpu.sync_copy(src_hbm.at[w, st], srcv)
        pltpu.sync_copy(dst_hbm.at[w, st], dstv)
        _gather(0, rows_a, sem_a)
        def _pair(k, carry2):
            _gwait(2 * k, rows_a, sem_a)
            _gather(2 * k + 1, rows_b, sem_a)
            _scat(2 * k, rows_a)
            _gwait(2 * k + 1, rows_b, sem_a)
            _gather(2 * k + 2, rows_a, sem_a)
            _scat(2 * k + 1, rows_b)
            return carry2
        lax.fori_loop(0, (_CPS - 1) // 2, _pair, 0)
        _gwait(_CPS - 1, rows_a, sem_a)
        _scat(_CPS - 1, rows_a)
        return carry
    lax.fori_loop(0, _NSTG, _stage, 0)
    plsc.subcore_barrier()

    pltpu.sync_copy(acc.at[pl.ds(s * _RPS, _RPS)],
                    out_hbm.at[c, pl.ds(s * _RPS, _RPS)])

    @pl.when(s == _NS - 1)
    def _():
        pltpu.sync_copy(acc.at[pl.ds(_RPS * _NS, _TAIL)],
                        out_hbm.at[c, pl.ds(_RPS * _NS, _TAIL)])

    if with_deg:
        pltpu.sync_copy(dacc.at[pl.ds(s * _DPS, _DPS)],
                        deg_hbm.at[c, pl.ds(s * _DPS, _DPS)])


def _make_sc_agg(with_deg):
    mesh = plsc.VectorSubcoreMesh(core_axis_name="c", subcore_axis_name="s")
    out_type = [jax.ShapeDtypeStruct((_NC, _N, _D), jnp.float32)]
    if with_deg:
        out_type.append(jax.ShapeDtypeStruct((_NC, _DEG_PAD), jnp.float32))
    scratch = [
        pltpu.VMEM((_CPS, _CH), jnp.int32),
        pltpu.VMEM((_CPS, _CH), jnp.int32),
        pltpu.VMEM((_CH, _D), jnp.float32),
        pltpu.VMEM((_CH, _D), jnp.float32),
        pltpu.VMEM((_CH,), jnp.float32),
        pltpu.VMEM_SHARED((_N, _D), jnp.float32),
    ]
    if with_deg:
        scratch.append(pltpu.VMEM_SHARED((_DEG_PAD,), jnp.float32))
    scratch.append(pltpu.SemaphoreType.DMA)
    scratch.append(pltpu.SemaphoreType.DMA)
    return pl.kernel(
        functools.partial(_sc_agg_body, with_deg),
        out_type=tuple(out_type) if with_deg else out_type[0],
        mesh=mesh,
        scratch_types=scratch,
    )


_R = 2000


def _layer1_body(h_ref, agg_ref, deg_ref, ws_ref, wn_ref, b_ref, out_ref):
    a = agg_ref[0] + agg_ref[1]
    inv = 1.0 / jnp.maximum(deg_ref[...], 1.0)
    out_ref[...] = jnp.maximum(
        jnp.dot(h_ref[...], ws_ref[...], preferred_element_type=jnp.float32)
        + jnp.dot(a * inv, wn_ref[...], preferred_element_type=jnp.float32)
        + b_ref[...], 0.0)


def _layer2_body(h_ref, agg_ref, deg_ref, ws_ref, wn_ref, b_ref, wp_ref,
                 bp_ref, out_ref, colsum):
    i = pl.program_id(0)
    a = agg_ref[0] + agg_ref[1]
    inv = 1.0 / jnp.maximum(deg_ref[...], 1.0)
    h2 = jnp.maximum(
        jnp.dot(h_ref[...], ws_ref[...], preferred_element_type=jnp.float32)
        + jnp.dot(a * inv, wn_ref[...], preferred_element_type=jnp.float32)
        + b_ref[...], 0.0)

    @pl.when(i == 0)
    def _():
        colsum[...] = jnp.zeros_like(colsum)

    colsum[...] += jnp.sum(h2, axis=0, keepdims=True)

    @pl.when(i == pl.num_programs(0) - 1)
    def _():
        out_ref[...] = (
            jnp.dot(colsum[...] * (1.0 / _N), wp_ref[...],
                    preferred_element_type=jnp.float32) + bp_ref[...])


def _tc_layer1(h, agg, deg2d, Ws, Wn, b):
    return pl.pallas_call(
        _layer1_body,
        grid=(_N // _R,),
        in_specs=[
            pl.BlockSpec((_R, _D), lambda i: (i, 0)),
            pl.BlockSpec((_NC, _R, _D), lambda i: (0, i, 0)),
            pl.BlockSpec((_R, 1), lambda i: (i, 0)),
            pl.BlockSpec((_D, _D), lambda i: (0, 0)),
            pl.BlockSpec((_D, _D), lambda i: (0, 0)),
            pl.BlockSpec((1, _D), lambda i: (0, 0)),
        ],
        out_specs=pl.BlockSpec((_R, _D), lambda i: (i, 0)),
        out_shape=jax.ShapeDtypeStruct((_N, _D), jnp.float32),
    )(h, agg, deg2d, Ws, Wn, b)


def _tc_layer2(h, agg, deg2d, Ws, Wn, b, Wp, bp):
    return pl.pallas_call(
        _layer2_body,
        grid=(_N // _R,),
        in_specs=[
            pl.BlockSpec((_R, _D), lambda i: (i, 0)),
            pl.BlockSpec((_NC, _R, _D), lambda i: (0, i, 0)),
            pl.BlockSpec((_R, 1), lambda i: (i, 0)),
            pl.BlockSpec((_D, _D), lambda i: (0, 0)),
            pl.BlockSpec((_D, _D), lambda i: (0, 0)),
            pl.BlockSpec((1, _D), lambda i: (0, 0)),
            pl.BlockSpec((_D, 1), lambda i: (0, 0)),
            pl.BlockSpec((1, 1), lambda i: (0, 0)),
        ],
        out_specs=pl.BlockSpec((1, 1), lambda i: (0, 0)),
        out_shape=jax.ShapeDtypeStruct((1, 1), jnp.float32),
        scratch_shapes=[pltpu.VMEM((1, _D), jnp.float32)],
    )(h, agg, deg2d, Ws, Wn, b, Wp, bp)


def kernel(x, edge_index, W_self1, W_neigh1, b1, W_self2, W_neigh2, b2, Wp, bp):
    src = edge_index[0].astype(jnp.int32).reshape(_NW, _NSTG, _CPS, _CH)
    dst = edge_index[1].astype(jnp.int32).reshape(_NW, _NSTG, _CPS, _CH)

    agg1, degp = _make_sc_agg(True)(x, src, dst)
    deg2d = (degp[0, :_N] + degp[1, :_N]).reshape(_N, 1)
    h1 = _tc_layer1(x, agg1, deg2d, W_self1, W_neigh1, b1.reshape(1, _D))
    agg2 = _make_sc_agg(False)(h1, src, dst)
    return _tc_layer2(h1, agg2, deg2d, W_self2, W_neigh2, b2.reshape(1, _D),
                      Wp, bp.reshape(1, 1))


# speedup vs baseline: 9.6808x; 1.0588x over previous
import functools

import jax
import jax.numpy as jnp
from jax import lax
from jax.experimental import pallas as pl
from jax.experimental.pallas import tpu as pltpu
from jax.experimental.pallas import tpu_sc as plsc

_N = 10000
_E = 320000
_D = 128

_NC = 2
_NS = 16
_NW = _NC * _NS
_CH = 128
_EP = 327680
_EPW = _EP // _NW
_NSTG = 10
_CPS = 8
_NPAD = 240
_NA = _N + _NPAD
_RPS = 624
_TAIL = _N - _RPS * _NS
_DEG_PAD = 10240
_DPS = _DEG_PAD // _NS


def _sc_agg_body(with_deg, h_hbm, src_hbm, dst_hbm, *refs):
    if with_deg:
        (out_hbm, deg_hbm, srcv, dstv, rows_a, rows_b, onesv, acc, dacc,
         sem_a, sem_b) = refs
    else:
        (out_hbm, srcv, dstv, rows_a, rows_b, onesv, acc, sem_a, sem_b) = refs
    c = lax.axis_index("c")
    s = lax.axis_index("s")
    w = s * _NC + c

    zero16 = jnp.zeros((16,), jnp.float32)
    def _zrow(i, carry):
        rows_a[i // (_D // 16), pl.ds((i % (_D // 16)) * 16, 16)] = zero16
        return carry
    lax.fori_loop(0, _CH * _D // 16, _zrow, 0)
    one16 = jnp.full((16,), 1.0, jnp.float32)
    for i in range(_CH // 16):
        onesv[pl.ds(i * 16, 16)] = one16

    for t in range(_RPS // _CH):
        pltpu.sync_copy(rows_a, acc.at[pl.ds(s * _RPS + t * _CH, _CH)])
    rem = _RPS % _CH
    if rem:
        pltpu.sync_copy(rows_a.at[pl.ds(0, rem)],
                        acc.at[pl.ds(s * _RPS + (_RPS // _CH) * _CH, rem)])

    @pl.when(s == _NS - 1)
    def _():
        pltpu.sync_copy(rows_a.at[pl.ds(0, _TAIL)],
                        acc.at[pl.ds(_RPS * _NS, _TAIL)])

    if with_deg:
        for t in range(_DPS // _D):
            pltpu.sync_copy(rows_a.at[0], dacc.at[pl.ds(s * _DPS + t * _D, _D)])
    plsc.subcore_barrier()

    def _gather(j, buf, sem):
        pltpu.async_copy(h_hbm.at[srcv.at[j]], buf, sem)

    def _gwait(j, buf, sem):
        pltpu.make_async_copy(h_hbm.at[srcv.at[j]], buf, sem).wait()

    def _scat(j, buf):
        pltpu.sync_copy(buf, acc.at[dstv.at[j]], add=True)
        if with_deg:
            pltpu.sync_copy(onesv, dacc.at[dstv.at[j]], add=True)

    def _stage(st, carry):
        pltpu.sync_copy(src_hbm.at[w, st], srcv)
        pltpu.sync_copy(dst_hbm.at[w, st], dstv)
        _gather(0, rows_a, sem_a)
        def _pair(k, carry2):
            _gwait(2 * k, rows_a, sem_a)
            _gather(2 * k + 1, rows_b, sem_a)
            _scat(2 * k, rows_a)
            _gwait(2 * k + 1, rows_b, sem_a)
            _gather(2 * k + 2, rows_a, sem_a)
            _scat(2 * k + 1, rows_b)
            return carry2
        lax.fori_loop(0, _CPS // 2 - 1, _pair, 0)
        _gwait(_CPS - 2, rows_a, sem_a)
        _gather(_CPS - 1, rows_b, sem_a)
        _scat(_CPS - 2, rows_a)
        _gwait(_CPS - 1, rows_b, sem_a)
        _scat(_CPS - 1, rows_b)
        return carry
    lax.fori_loop(0, _NSTG, _stage, 0)
    plsc.subcore_barrier()

    pltpu.sync_copy(acc.at[pl.ds(s * _RPS, _RPS)],
                    out_hbm.at[c, pl.ds(s * _RPS, _RPS)])

    @pl.when(s == _NS - 1)
    def _():
        pltpu.sync_copy(acc.at[pl.ds(_RPS * _NS, _TAIL)],
                        out_hbm.at[c, pl.ds(_RPS * _NS, _TAIL)])

    if with_deg:
        pltpu.sync_copy(dacc.at[pl.ds(s * _DPS, _DPS)],
                        deg_hbm.at[c, pl.ds(s * _DPS, _DPS)])


def _make_sc_agg(with_deg):
    mesh = plsc.VectorSubcoreMesh(core_axis_name="c", subcore_axis_name="s")
    out_type = [jax.ShapeDtypeStruct((_NC, _N, _D), jnp.float32)]
    if with_deg:
        out_type.append(jax.ShapeDtypeStruct((_NC, _DEG_PAD), jnp.float32))
    scratch = [
        pltpu.VMEM((_CPS, _CH), jnp.int32),
        pltpu.VMEM((_CPS, _CH), jnp.int32),
        pltpu.VMEM((_CH, _D), jnp.float32),
        pltpu.VMEM((_CH, _D), jnp.float32),
        pltpu.VMEM((_CH,), jnp.float32),
        pltpu.VMEM_SHARED((_NA, _D), jnp.float32),
    ]
    if with_deg:
        scratch.append(pltpu.VMEM_SHARED((_DEG_PAD,), jnp.float32))
    scratch.append(pltpu.SemaphoreType.DMA)
    scratch.append(pltpu.SemaphoreType.DMA)
    return pl.kernel(
        functools.partial(_sc_agg_body, with_deg),
        out_type=tuple(out_type) if with_deg else out_type[0],
        mesh=mesh,
        scratch_types=scratch,
    )


_R = 2000


def _layer1_body(h_ref, agg_ref, deg_ref, ws_ref, wn_ref, b_ref, out_ref):
    a = agg_ref[0] + agg_ref[1]
    inv = 1.0 / jnp.maximum(deg_ref[...], 1.0)
    out_ref[...] = jnp.maximum(
        jnp.dot(h_ref[...], ws_ref[...], preferred_element_type=jnp.float32, precision=jax.lax.Precision.HIGHEST)
        + jnp.dot(a * inv, wn_ref[...], preferred_element_type=jnp.float32, precision=jax.lax.Precision.HIGHEST)
        + b_ref[...], 0.0)


def _layer2_body(h_ref, agg_ref, deg_ref, ws_ref, wn_ref, b_ref, wp_ref,
                 bp_ref, out_ref, colsum):
    i = pl.program_id(0)
    a = agg_ref[0] + agg_ref[1]
    inv = 1.0 / jnp.maximum(deg_ref[...], 1.0)
    h2 = jnp.maximum(
        jnp.dot(h_ref[...], ws_ref[...], preferred_element_type=jnp.float32, precision=jax.lax.Precision.HIGHEST)
        + jnp.dot(a * inv, wn_ref[...], preferred_element_type=jnp.float32, precision=jax.lax.Precision.HIGHEST)
        + b_ref[...], 0.0)

    @pl.when(i == 0)
    def _():
        colsum[...] = jnp.zeros_like(colsum)

    colsum[...] += jnp.sum(h2, axis=0, keepdims=True)

    @pl.when(i == pl.num_programs(0) - 1)
    def _():
        out_ref[...] = (
            jnp.dot(colsum[...] * (1.0 / _N), wp_ref[...],
                    preferred_element_type=jnp.float32, precision=jax.lax.Precision.HIGHEST) + bp_ref[...])


def _tc_layer1(h, agg, deg2d, Ws, Wn, b):
    return pl.pallas_call(
        _layer1_body,
        grid=(_N // _R,),
        in_specs=[
            pl.BlockSpec((_R, _D), lambda i: (i, 0)),
            pl.BlockSpec((_NC, _R, _D), lambda i: (0, i, 0)),
            pl.BlockSpec((_R, 1), lambda i: (i, 0)),
            pl.BlockSpec((_D, _D), lambda i: (0, 0)),
            pl.BlockSpec((_D, _D), lambda i: (0, 0)),
            pl.BlockSpec((1, _D), lambda i: (0, 0)),
        ],
        out_specs=pl.BlockSpec((_R, _D), lambda i: (i, 0)),
        out_shape=jax.ShapeDtypeStruct((_N, _D), jnp.float32),
    )(h, agg, deg2d, Ws, Wn, b)


def _tc_layer2(h, agg, deg2d, Ws, Wn, b, Wp, bp):
    return pl.pallas_call(
        _layer2_body,
        grid=(_N // _R,),
        in_specs=[
            pl.BlockSpec((_R, _D), lambda i: (i, 0)),
            pl.BlockSpec((_NC, _R, _D), lambda i: (0, i, 0)),
            pl.BlockSpec((_R, 1), lambda i: (i, 0)),
            pl.BlockSpec((_D, _D), lambda i: (0, 0)),
            pl.BlockSpec((_D, _D), lambda i: (0, 0)),
            pl.BlockSpec((1, _D), lambda i: (0, 0)),
            pl.BlockSpec((_D, 1), lambda i: (0, 0)),
            pl.BlockSpec((1, 1), lambda i: (0, 0)),
        ],
        out_specs=pl.BlockSpec((1, 1), lambda i: (0, 0)),
        out_shape=jax.ShapeDtypeStruct((1, 1), jnp.float32),
        scratch_shapes=[pltpu.VMEM((1, _D), jnp.float32)],
    )(h, agg, deg2d, Ws, Wn, b, Wp, bp)


def kernel(x, edge_index, W_self1, W_neigh1, b1, W_self2, W_neigh2, b2, Wp, bp):
    npad = _EP - _E
    pad_iota = jnp.arange(npad, dtype=jnp.int32)
    src = jnp.concatenate(
        [edge_index[0].astype(jnp.int32), pad_iota % _N]
    ).reshape(_NW, _NSTG, _CPS, _CH)
    dst = jnp.concatenate(
        [edge_index[1].astype(jnp.int32), _N + pad_iota % _NPAD]
    ).reshape(_NW, _NSTG, _CPS, _CH)

    agg1, degp = _make_sc_agg(True)(x, src, dst)
    deg2d = (degp[0, :_N] + degp[1, :_N]).reshape(_N, 1)
    h1 = _tc_layer1(x, agg1, deg2d, W_self1, W_neigh1, b1.reshape(1, _D))
    agg2 = _make_sc_agg(False)(h1, src, dst)
    return _tc_layer2(h1, agg2, deg2d, W_self2, W_neigh2, b2.reshape(1, _D),
                      Wp, bp.reshape(1, 1))


# speedup vs baseline: 9.7392x; 1.0060x over previous
import functools

import jax
import jax.numpy as jnp
from jax import lax
from jax.experimental import pallas as pl
from jax.experimental.pallas import tpu as pltpu
from jax.experimental.pallas import tpu_sc as plsc

_N = 10000
_E = 320000
_D = 128

_NC = 2
_NS = 16
_NW = _NC * _NS
_CH = 128
_EP = 327680
_EPW = _EP // _NW
_NSTG = 10
_CPS = 8
_NPAD = 240
_NA = _N + _NPAD
_RPS = 624
_TAIL = _N - _RPS * _NS
_DEG_PAD = 10240
_DPS = _DEG_PAD // _NS


def _sc_agg_body(with_deg, h_hbm, src_hbm, dst_hbm, *refs):
    if with_deg:
        (out_hbm, deg_hbm, srcv, dstv, rows_a, rows_b, onesv, acc, dacc,
         sem_a, sem_b) = refs
    else:
        (out_hbm, srcv, dstv, rows_a, rows_b, onesv, acc, sem_a, sem_b) = refs
    c = lax.axis_index("c")
    s = lax.axis_index("s")
    w = s * _NC + c

    zero16 = jnp.zeros((16,), jnp.float32)
    def _zrow(i, carry):
        rows_a[i // (_D // 16), pl.ds((i % (_D // 16)) * 16, 16)] = zero16
        return carry
    lax.fori_loop(0, _CH * _D // 16, _zrow, 0)
    one16 = jnp.full((16,), 1.0, jnp.float32)
    for i in range(_CH // 16):
        onesv[pl.ds(i * 16, 16)] = one16

    for t in range(_RPS // _CH):
        pltpu.sync_copy(rows_a, acc.at[pl.ds(s * _RPS + t * _CH, _CH)])
    rem = _RPS % _CH
    if rem:
        pltpu.sync_copy(rows_a.at[pl.ds(0, rem)],
                        acc.at[pl.ds(s * _RPS + (_RPS // _CH) * _CH, rem)])

    @pl.when(s == _NS - 1)
    def _():
        pltpu.sync_copy(rows_a.at[pl.ds(0, _TAIL)],
                        acc.at[pl.ds(_RPS * _NS, _TAIL)])

    if with_deg:
        for t in range(_DPS // _D):
            pltpu.sync_copy(rows_a.at[0], dacc.at[pl.ds(s * _DPS + t * _D, _D)])
    plsc.subcore_barrier()

    def _g(j, buf):
        pltpu.async_copy(h_hbm.at[srcv.at[j]], buf, sem_a)

    def _gw(j, buf):
        pltpu.make_async_copy(h_hbm.at[srcv.at[j]], buf, sem_a).wait()

    def _s(j, buf):
        pltpu.async_copy(buf, acc.at[dstv.at[j]], sem_b, add=True)
        if with_deg:
            pltpu.sync_copy(onesv, dacc.at[dstv.at[j]], add=True)

    def _sw(j, buf):
        pltpu.make_async_copy(buf, acc.at[dstv.at[j]], sem_b).wait()

    def _stage(st, carry):
        pltpu.sync_copy(src_hbm.at[w, st], srcv)
        pltpu.sync_copy(dst_hbm.at[w, st], dstv)
        _g(0, rows_a)
        _gw(0, rows_a)
        _g(1, rows_b)
        _s(0, rows_a)
        def _pair(k, carry2):
            _gw(2 * k - 1, rows_b)
            _sw(2 * k - 2, rows_a)
            _g(2 * k, rows_a)
            _s(2 * k - 1, rows_b)
            _gw(2 * k, rows_a)
            _sw(2 * k - 1, rows_b)
            _g(2 * k + 1, rows_b)
            _s(2 * k, rows_a)
            return carry2
        lax.fori_loop(1, _CPS // 2, _pair, 0)
        _gw(_CPS - 1, rows_b)
        _sw(_CPS - 2, rows_a)
        _s(_CPS - 1, rows_b)
        _sw(_CPS - 1, rows_b)
        return carry
    lax.fori_loop(0, _NSTG, _stage, 0)
    plsc.subcore_barrier()

    pltpu.sync_copy(acc.at[pl.ds(s * _RPS, _RPS)],
                    out_hbm.at[c, pl.ds(s * _RPS, _RPS)])

    @pl.when(s == _NS - 1)
    def _():
        pltpu.sync_copy(acc.at[pl.ds(_RPS * _NS, _TAIL)],
                        out_hbm.at[c, pl.ds(_RPS * _NS, _TAIL)])

    if with_deg:
        pltpu.sync_copy(dacc.at[pl.ds(s * _DPS, _DPS)],
                        deg_hbm.at[c, pl.ds(s * _DPS, _DPS)])


def _make_sc_agg(with_deg):
    mesh = plsc.VectorSubcoreMesh(core_axis_name="c", subcore_axis_name="s")
    out_type = [jax.ShapeDtypeStruct((_NC, _N, _D), jnp.float32)]
    if with_deg:
        out_type.append(jax.ShapeDtypeStruct((_NC, _DEG_PAD), jnp.float32))
    scratch = [
        pltpu.VMEM((_CPS, _CH), jnp.int32),
        pltpu.VMEM((_CPS, _CH), jnp.int32),
        pltpu.VMEM((_CH, _D), jnp.float32),
        pltpu.VMEM((_CH, _D), jnp.float32),
        pltpu.VMEM((_CH,), jnp.float32),
        pltpu.VMEM_SHARED((_NA, _D), jnp.float32),
    ]
    if with_deg:
        scratch.append(pltpu.VMEM_SHARED((_DEG_PAD,), jnp.float32))
    scratch.append(pltpu.SemaphoreType.DMA)
    scratch.append(pltpu.SemaphoreType.DMA)
    return pl.kernel(
        functools.partial(_sc_agg_body, with_deg),
        out_type=tuple(out_type) if with_deg else out_type[0],
        mesh=mesh,
        scratch_types=scratch,
    )


_R = 2000


def _layer1_body(h_ref, agg_ref, deg_ref, ws_ref, wn_ref, b_ref, out_ref):
    a = agg_ref[0] + agg_ref[1]
    inv = 1.0 / jnp.maximum(deg_ref[...], 1.0)
    out_ref[...] = jnp.maximum(
        jnp.dot(h_ref[...], ws_ref[...], preferred_element_type=jnp.float32, precision=jax.lax.Precision.HIGHEST)
        + jnp.dot(a * inv, wn_ref[...], preferred_element_type=jnp.float32, precision=jax.lax.Precision.HIGHEST)
        + b_ref[...], 0.0)


def _layer2_body(h_ref, agg_ref, deg_ref, ws_ref, wn_ref, b_ref, wp_ref,
                 bp_ref, out_ref, colsum):
    i = pl.program_id(0)
    a = agg_ref[0] + agg_ref[1]
    inv = 1.0 / jnp.maximum(deg_ref[...], 1.0)
    h2 = jnp.maximum(
        jnp.dot(h_ref[...], ws_ref[...], preferred_element_type=jnp.float32, precision=jax.lax.Precision.HIGHEST)
        + jnp.dot(a * inv, wn_ref[...], preferred_element_type=jnp.float32, precision=jax.lax.Precision.HIGHEST)
        + b_ref[...], 0.0)

    @pl.when(i == 0)
    def _():
        colsum[...] = jnp.zeros_like(colsum)

    colsum[...] += jnp.sum(h2, axis=0, keepdims=True)

    @pl.when(i == pl.num_programs(0) - 1)
    def _():
        out_ref[...] = (
            jnp.dot(colsum[...] * (1.0 / _N), wp_ref[...],
                    preferred_element_type=jnp.float32, precision=jax.lax.Precision.HIGHEST) + bp_ref[...])


def _tc_layer1(h, agg, deg2d, Ws, Wn, b):
    return pl.pallas_call(
        _layer1_body,
        grid=(_N // _R,),
        in_specs=[
            pl.BlockSpec((_R, _D), lambda i: (i, 0)),
            pl.BlockSpec((_NC, _R, _D), lambda i: (0, i, 0)),
            pl.BlockSpec((_R, 1), lambda i: (i, 0)),
            pl.BlockSpec((_D, _D), lambda i: (0, 0)),
            pl.BlockSpec((_D, _D), lambda i: (0, 0)),
            pl.BlockSpec((1, _D), lambda i: (0, 0)),
        ],
        out_specs=pl.BlockSpec((_R, _D), lambda i: (i, 0)),
        out_shape=jax.ShapeDtypeStruct((_N, _D), jnp.float32),
    )(h, agg, deg2d, Ws, Wn, b)


def _tc_layer2(h, agg, deg2d, Ws, Wn, b, Wp, bp):
    return pl.pallas_call(
        _layer2_body,
        grid=(_N // _R,),
        in_specs=[
            pl.BlockSpec((_R, _D), lambda i: (i, 0)),
            pl.BlockSpec((_NC, _R, _D), lambda i: (0, i, 0)),
            pl.BlockSpec((_R, 1), lambda i: (i, 0)),
            pl.BlockSpec((_D, _D), lambda i: (0, 0)),
            pl.BlockSpec((_D, _D), lambda i: (0, 0)),
            pl.BlockSpec((1, _D), lambda i: (0, 0)),
            pl.BlockSpec((_D, 1), lambda i: (0, 0)),
            pl.BlockSpec((1, 1), lambda i: (0, 0)),
        ],
        out_specs=pl.BlockSpec((1, 1), lambda i: (0, 0)),
        out_shape=jax.ShapeDtypeStruct((1, 1), jnp.float32),
        scratch_shapes=[pltpu.VMEM((1, _D), jnp.float32)],
    )(h, agg, deg2d, Ws, Wn, b, Wp, bp)


def kernel(x, edge_index, W_self1, W_neigh1, b1, W_self2, W_neigh2, b2, Wp, bp):
    npad = _EP - _E
    pad_iota = jnp.arange(npad, dtype=jnp.int32)
    src = jnp.concatenate(
        [edge_index[0].astype(jnp.int32), pad_iota % _N]
    ).reshape(_NW, _NSTG, _CPS, _CH)
    dst = jnp.concatenate(
        [edge_index[1].astype(jnp.int32), _N + pad_iota % _NPAD]
    ).reshape(_NW, _NSTG, _CPS, _CH)

    agg1, degp = _make_sc_agg(True)(x, src, dst)
    deg2d = (degp[0, :_N] + degp[1, :_N]).reshape(_N, 1)
    h1 = _tc_layer1(x, agg1, deg2d, W_self1, W_neigh1, b1.reshape(1, _D))
    agg2 = _make_sc_agg(False)(h1, src, dst)
    return _tc_layer2(h1, agg2, deg2d, W_self2, W_neigh2, b2.reshape(1, _D),
                      Wp, bp.reshape(1, 1))


# speedup vs baseline: 10.2904x; 1.0566x over previous
import functools

import jax
import jax.numpy as jnp
from jax import lax
from jax.experimental import pallas as pl
from jax.experimental.pallas import tpu as pltpu
from jax.experimental.pallas import tpu_sc as plsc

_N = 10000
_E = 320000
_D = 128

_NC = 2
_NS = 16
_NW = _NC * _NS
_CH = 128
_EP = 327680
_EPW = _EP // _NW
_NSTG = 10
_CPS = 8
_NPAD = 240
_NA = _N + _NPAD
_RPS = 624
_TAIL = _N - _RPS * _NS
_DEG_PAD = 10240
_DPS = _DEG_PAD // _NS


def _sc_agg_body(with_deg, h_hbm, src_hbm, dst_hbm, *refs):
    if with_deg:
        (out_hbm, deg_hbm, srcv, dstv, rows_a, rows_b, onesv, acc, dacc,
         sem_a, sem_b, sem_i) = refs
    else:
        (out_hbm, srcv, dstv, rows_a, rows_b, onesv, acc, sem_a, sem_b,
         sem_i) = refs
    c = lax.axis_index("c")
    s = lax.axis_index("s")
    w = s * _NC + c

    zero16 = jnp.zeros((16,), jnp.float32)
    def _zrow(i, carry):
        rows_a[i // (_D // 16), pl.ds((i % (_D // 16)) * 16, 16)] = zero16
        return carry
    lax.fori_loop(0, _CH * _D // 16, _zrow, 0)
    one16 = jnp.full((16,), 1.0, jnp.float32)
    for i in range(_CH // 16):
        onesv[pl.ds(i * 16, 16)] = one16

    for t in range(_RPS // _CH):
        pltpu.sync_copy(rows_a, acc.at[pl.ds(s * _RPS + t * _CH, _CH)])
    rem = _RPS % _CH
    if rem:
        pltpu.sync_copy(rows_a.at[pl.ds(0, rem)],
                        acc.at[pl.ds(s * _RPS + (_RPS // _CH) * _CH, rem)])

    @pl.when(s == _NS - 1)
    def _():
        pltpu.sync_copy(rows_a.at[pl.ds(0, _TAIL)],
                        acc.at[pl.ds(_RPS * _NS, _TAIL)])

    if with_deg:
        for t in range(_DPS // _D):
            pltpu.sync_copy(rows_a.at[0], dacc.at[pl.ds(s * _DPS + t * _D, _D)])
    plsc.subcore_barrier()

    def _g(p, j, buf):
        pltpu.async_copy(h_hbm.at[srcv.at[p, j]], buf, sem_a)

    def _gw(p, j, buf):
        pltpu.make_async_copy(h_hbm.at[srcv.at[p, j]], buf, sem_a).wait()

    def _s(p, j, buf):
        pltpu.async_copy(buf, acc.at[dstv.at[p, j]], sem_b, add=True)
        if with_deg:
            pltpu.sync_copy(onesv, dacc.at[dstv.at[p, j]], add=True)

    def _sw(p, j, buf):
        pltpu.make_async_copy(buf, acc.at[dstv.at[p, j]], sem_b).wait()

    pltpu.sync_copy(src_hbm.at[w, 0], srcv.at[0])
    pltpu.sync_copy(dst_hbm.at[w, 0], dstv.at[0])

    def _stage(st, carry):
        p = st % 2

        @pl.when(st < _NSTG - 1)
        def _():
            pltpu.async_copy(src_hbm.at[w, st + 1], srcv.at[1 - p], sem_i)
            pltpu.async_copy(dst_hbm.at[w, st + 1], dstv.at[1 - p], sem_i)

        _g(p, 0, rows_a)
        _gw(p, 0, rows_a)
        _g(p, 1, rows_b)
        _s(p, 0, rows_a)
        def _pair(k, carry2):
            _gw(p, 2 * k - 1, rows_b)
            _sw(p, 2 * k - 2, rows_a)
            _g(p, 2 * k, rows_a)
            _s(p, 2 * k - 1, rows_b)
            _gw(p, 2 * k, rows_a)
            _sw(p, 2 * k - 1, rows_b)
            _g(p, 2 * k + 1, rows_b)
            _s(p, 2 * k, rows_a)
            return carry2
        lax.fori_loop(1, _CPS // 2, _pair, 0)
        _gw(p, _CPS - 1, rows_b)
        _sw(p, _CPS - 2, rows_a)
        _s(p, _CPS - 1, rows_b)
        _sw(p, _CPS - 1, rows_b)

        @pl.when(st < _NSTG - 1)
        def _():
            pltpu.make_async_copy(src_hbm.at[w, st + 1], srcv.at[1 - p],
                                  sem_i).wait()
            pltpu.make_async_copy(dst_hbm.at[w, st + 1], dstv.at[1 - p],
                                  sem_i).wait()
        return carry
    lax.fori_loop(0, _NSTG, _stage, 0)
    plsc.subcore_barrier()

    pltpu.sync_copy(acc.at[pl.ds(s * _RPS, _RPS)],
                    out_hbm.at[c, pl.ds(s * _RPS, _RPS)])

    @pl.when(s == _NS - 1)
    def _():
        pltpu.sync_copy(acc.at[pl.ds(_RPS * _NS, _TAIL)],
                        out_hbm.at[c, pl.ds(_RPS * _NS, _TAIL)])

    if with_deg:
        pltpu.sync_copy(dacc.at[pl.ds(s * _DPS, _DPS)],
                        deg_hbm.at[c, pl.ds(s * _DPS, _DPS)])


def _make_sc_agg(with_deg):
    mesh = plsc.VectorSubcoreMesh(core_axis_name="c", subcore_axis_name="s")
    out_type = [jax.ShapeDtypeStruct((_NC, _N, _D), jnp.float32)]
    if with_deg:
        out_type.append(jax.ShapeDtypeStruct((_NC, _DEG_PAD), jnp.float32))
    scratch = [
        pltpu.VMEM((2, _CPS, _CH), jnp.int32),
        pltpu.VMEM((2, _CPS, _CH), jnp.int32),
        pltpu.VMEM((_CH, _D), jnp.float32),
        pltpu.VMEM((_CH, _D), jnp.float32),
        pltpu.VMEM((_CH,), jnp.float32),
        pltpu.VMEM_SHARED((_NA, _D), jnp.float32),
    ]
    if with_deg:
        scratch.append(pltpu.VMEM_SHARED((_DEG_PAD,), jnp.float32))
    scratch.append(pltpu.SemaphoreType.DMA)
    scratch.append(pltpu.SemaphoreType.DMA)
    scratch.append(pltpu.SemaphoreType.DMA)
    return pl.kernel(
        functools.partial(_sc_agg_body, with_deg),
        out_type=tuple(out_type) if with_deg else out_type[0],
        mesh=mesh,
        scratch_types=scratch,
    )


_R = 2000


def _layer1_body(h_ref, agg_ref, deg_ref, ws_ref, wn_ref, b_ref, out_ref):
    a = agg_ref[0] + agg_ref[1]
    inv = 1.0 / jnp.maximum(deg_ref[...], 1.0)
    out_ref[...] = jnp.maximum(
        jnp.dot(h_ref[...], ws_ref[...], preferred_element_type=jnp.float32, precision=jax.lax.Precision.HIGHEST)
        + jnp.dot(a * inv, wn_ref[...], preferred_element_type=jnp.float32, precision=jax.lax.Precision.HIGHEST)
        + b_ref[...], 0.0)


def _layer2_body(h_ref, agg_ref, deg_ref, ws_ref, wn_ref, b_ref, wp_ref,
                 bp_ref, out_ref, colsum):
    i = pl.program_id(0)
    a = agg_ref[0] + agg_ref[1]
    inv = 1.0 / jnp.maximum(deg_ref[...], 1.0)
    h2 = jnp.maximum(
        jnp.dot(h_ref[...], ws_ref[...], preferred_element_type=jnp.float32, precision=jax.lax.Precision.HIGHEST)
        + jnp.dot(a * inv, wn_ref[...], preferred_element_type=jnp.float32, precision=jax.lax.Precision.HIGHEST)
        + b_ref[...], 0.0)

    @pl.when(i == 0)
    def _():
        colsum[...] = jnp.zeros_like(colsum)

    colsum[...] += jnp.sum(h2, axis=0, keepdims=True)

    @pl.when(i == pl.num_programs(0) - 1)
    def _():
        out_ref[...] = (
            jnp.dot(colsum[...] * (1.0 / _N), wp_ref[...],
                    preferred_element_type=jnp.float32, precision=jax.lax.Precision.HIGHEST) + bp_ref[...])


def _tc_layer1(h, agg, deg2d, Ws, Wn, b):
    return pl.pallas_call(
        _layer1_body,
        grid=(_N // _R,),
        in_specs=[
            pl.BlockSpec((_R, _D), lambda i: (i, 0)),
            pl.BlockSpec((_NC, _R, _D), lambda i: (0, i, 0)),
            pl.BlockSpec((_R, 1), lambda i: (i, 0)),
            pl.BlockSpec((_D, _D), lambda i: (0, 0)),
            pl.BlockSpec((_D, _D), lambda i: (0, 0)),
            pl.BlockSpec((1, _D), lambda i: (0, 0)),
        ],
        out_specs=pl.BlockSpec((_R, _D), lambda i: (i, 0)),
        out_shape=jax.ShapeDtypeStruct((_N, _D), jnp.float32),
    )(h, agg, deg2d, Ws, Wn, b)


def _tc_layer2(h, agg, deg2d, Ws, Wn, b, Wp, bp):
    return pl.pallas_call(
        _layer2_body,
        grid=(_N // _R,),
        in_specs=[
            pl.BlockSpec((_R, _D), lambda i: (i, 0)),
            pl.BlockSpec((_NC, _R, _D), lambda i: (0, i, 0)),
            pl.BlockSpec((_R, 1), lambda i: (i, 0)),
            pl.BlockSpec((_D, _D), lambda i: (0, 0)),
            pl.BlockSpec((_D, _D), lambda i: (0, 0)),
            pl.BlockSpec((1, _D), lambda i: (0, 0)),
            pl.BlockSpec((_D, 1), lambda i: (0, 0)),
            pl.BlockSpec((1, 1), lambda i: (0, 0)),
        ],
        out_specs=pl.BlockSpec((1, 1), lambda i: (0, 0)),
        out_shape=jax.ShapeDtypeStruct((1, 1), jnp.float32),
        scratch_shapes=[pltpu.VMEM((1, _D), jnp.float32)],
    )(h, agg, deg2d, Ws, Wn, b, Wp, bp)


def kernel(x, edge_index, W_self1, W_neigh1, b1, W_self2, W_neigh2, b2, Wp, bp):
    npad = _EP - _E
    pad_iota = jnp.arange(npad, dtype=jnp.int32)
    src = jnp.concatenate(
        [edge_index[0].astype(jnp.int32), pad_iota % _N]
    ).reshape(_NW, _NSTG, _CPS, _CH)
    dst = jnp.concatenate(
        [edge_index[1].astype(jnp.int32), _N + pad_iota % _NPAD]
    ).reshape(_NW, _NSTG, _CPS, _CH)

    agg1, degp = _make_sc_agg(True)(x, src, dst)
    deg2d = (degp[0, :_N] + degp[1, :_N]).reshape(_N, 1)
    h1 = _tc_layer1(x, agg1, deg2d, W_self1, W_neigh1, b1.reshape(1, _D))
    agg2 = _make_sc_agg(False)(h1, src, dst)
    return _tc_layer2(h1, agg2, deg2d, W_self2, W_neigh2, b2.reshape(1, _D),
                      Wp, bp.reshape(1, 1))


# speedup vs baseline: 10.6399x; 1.0340x over previous
import functools

import jax
import jax.numpy as jnp
from jax import lax
from jax.experimental import pallas as pl
from jax.experimental.pallas import tpu as pltpu
from jax.experimental.pallas import tpu_sc as plsc

_N = 10000
_E = 320000
_D = 128

_NC = 2
_NS = 16
_NW = _NC * _NS
_CH = 128
_EP = 327680
_EPW = _EP // _NW
_NSTG = 10
_CPS = 8
_NPAD = 240
_NA = _N + _NPAD
_RPS = 624
_TAIL = _N - _RPS * _NS
_DEG_PAD = 10240
_DPS = _DEG_PAD // _NS


def _sc_agg_body(with_deg, h_hbm, src_hbm, dst_hbm, *refs):
    if with_deg:
        (out_hbm, deg_hbm, srcv, dstv, rows_a, rows_b, onesv, acc, dacc,
         sem_a, sem_b, sem_i) = refs
    else:
        (out_hbm, srcv, dstv, rows_a, rows_b, onesv, acc, sem_a, sem_b,
         sem_i) = refs
    c = lax.axis_index("c")
    s = lax.axis_index("s")
    w = s * _NC + c

    zero16 = jnp.zeros((16,), jnp.float32)
    def _zrow(i, carry):
        rows_a[i // (_D // 16), pl.ds((i % (_D // 16)) * 16, 16)] = zero16
        return carry
    lax.fori_loop(0, _CH * _D // 16, _zrow, 0)
    one16 = jnp.full((16,), 1.0, jnp.float32)
    for i in range(_CH // 16):
        onesv[pl.ds(i * 16, 16)] = one16

    for t in range(_RPS // _CH):
        pltpu.sync_copy(rows_a, acc.at[pl.ds(s * _RPS + t * _CH, _CH)])
    rem = _RPS % _CH
    if rem:
        pltpu.sync_copy(rows_a.at[pl.ds(0, rem)],
                        acc.at[pl.ds(s * _RPS + (_RPS // _CH) * _CH, rem)])

    @pl.when(s == _NS - 1)
    def _():
        pltpu.sync_copy(rows_a.at[pl.ds(0, _TAIL)],
                        acc.at[pl.ds(_RPS * _NS, _TAIL)])

    if with_deg:
        for t in range(_DPS // _D):
            pltpu.sync_copy(rows_a.at[0], dacc.at[pl.ds(s * _DPS + t * _D, _D)])
    plsc.subcore_barrier()

    def _sidx(j):
        return srcv.at[(j // _CPS) % 2, j % _CPS]

    def _didx(j):
        return dstv.at[(j // _CPS) % 2, j % _CPS]

    def _g(j, buf):
        pltpu.async_copy(h_hbm.at[_sidx(j)], buf, sem_a)

    def _gw(j, buf):
        pltpu.make_async_copy(h_hbm.at[_sidx(j)], buf, sem_a).wait()

    def _s(j, buf):
        pltpu.async_copy(buf, acc.at[_didx(j)], sem_b, add=True)
        if with_deg:
            pltpu.sync_copy(onesv, dacc.at[_didx(j)], add=True)

    def _sw(j, buf):
        pltpu.make_async_copy(buf, acc.at[_didx(j)], sem_b).wait()

    def _ipf(st):
        pltpu.async_copy(src_hbm.at[w, st], srcv.at[st % 2], sem_i)
        pltpu.async_copy(dst_hbm.at[w, st], dstv.at[st % 2], sem_i)

    def _ipw(st):
        pltpu.make_async_copy(src_hbm.at[w, st], srcv.at[st % 2],
                              sem_i).wait()
        pltpu.make_async_copy(dst_hbm.at[w, st], dstv.at[st % 2],
                              sem_i).wait()

    pltpu.sync_copy(src_hbm.at[w, 0], srcv.at[0])
    pltpu.sync_copy(dst_hbm.at[w, 0], dstv.at[0])
    _ipf(1)
    _g(0, rows_a)
    _gw(0, rows_a)
    _g(1, rows_b)
    _s(0, rows_a)

    _nchunk = _NSTG * _CPS
    def _pair(k, carry):
        j0 = 2 * k
        st = j0 // _CPS
        at_entry = (j0 % _CPS) == 0

        @pl.when(at_entry)
        def _():
            _ipw(st)
        _gw(j0 - 1, rows_b)
        _sw(j0 - 2, rows_a)
        _g(j0, rows_a)
        _s(j0 - 1, rows_b)
        _gw(j0, rows_a)
        _sw(j0 - 1, rows_b)

        @pl.when(at_entry & (st < _NSTG - 1))
        def _():
            _ipf(st + 1)
        _g(j0 + 1, rows_b)
        _s(j0, rows_a)
        return carry
    lax.fori_loop(1, _nchunk // 2, _pair, 0)
    _gw(_nchunk - 1, rows_b)
    _sw(_nchunk - 2, rows_a)
    _s(_nchunk - 1, rows_b)
    _sw(_nchunk - 1, rows_b)
    plsc.subcore_barrier()

    pltpu.sync_copy(acc.at[pl.ds(s * _RPS, _RPS)],
                    out_hbm.at[c, pl.ds(s * _RPS, _RPS)])

    @pl.when(s == _NS - 1)
    def _():
        pltpu.sync_copy(acc.at[pl.ds(_RPS * _NS, _TAIL)],
                        out_hbm.at[c, pl.ds(_RPS * _NS, _TAIL)])

    if with_deg:
        pltpu.sync_copy(dacc.at[pl.ds(s * _DPS, _DPS)],
                        deg_hbm.at[c, pl.ds(s * _DPS, _DPS)])


def _make_sc_agg(with_deg):
    mesh = plsc.VectorSubcoreMesh(core_axis_name="c", subcore_axis_name="s")
    out_type = [jax.ShapeDtypeStruct((_NC, _N, _D), jnp.float32)]
    if with_deg:
        out_type.append(jax.ShapeDtypeStruct((_NC, _DEG_PAD), jnp.float32))
    scratch = [
        pltpu.VMEM((2, _CPS, _CH), jnp.int32),
        pltpu.VMEM((2, _CPS, _CH), jnp.int32),
        pltpu.VMEM((_CH, _D), jnp.float32),
        pltpu.VMEM((_CH, _D), jnp.float32),
        pltpu.VMEM((_CH,), jnp.float32),
        pltpu.VMEM_SHARED((_NA, _D), jnp.float32),
    ]
    if with_deg:
        scratch.append(pltpu.VMEM_SHARED((_DEG_PAD,), jnp.float32))
    scratch.append(pltpu.SemaphoreType.DMA)
    scratch.append(pltpu.SemaphoreType.DMA)
    scratch.append(pltpu.SemaphoreType.DMA)
    return pl.kernel(
        functools.partial(_sc_agg_body, with_deg),
        out_type=tuple(out_type) if with_deg else out_type[0],
        mesh=mesh,
        scratch_types=scratch,
    )


_R = 2000


def _layer1_body(h_ref, agg_ref, deg_ref, ws_ref, wn_ref, b_ref, out_ref):
    a = agg_ref[0] + agg_ref[1]
    inv = 1.0 / jnp.maximum(deg_ref[...], 1.0)
    out_ref[...] = jnp.maximum(
        jnp.dot(h_ref[...], ws_ref[...], preferred_element_type=jnp.float32, precision=jax.lax.Precision.HIGHEST)
        + jnp.dot(a * inv, wn_ref[...], preferred_element_type=jnp.float32, precision=jax.lax.Precision.HIGHEST)
        + b_ref[...], 0.0)


def _layer2_body(h_ref, agg_ref, deg_ref, ws_ref, wn_ref, b_ref, wp_ref,
                 bp_ref, out_ref, colsum):
    i = pl.program_id(0)
    a = agg_ref[0] + agg_ref[1]
    inv = 1.0 / jnp.maximum(deg_ref[...], 1.0)
    h2 = jnp.maximum(
        jnp.dot(h_ref[...], ws_ref[...], preferred_element_type=jnp.float32, precision=jax.lax.Precision.HIGHEST)
        + jnp.dot(a * inv, wn_ref[...], preferred_element_type=jnp.float32, precision=jax.lax.Precision.HIGHEST)
        + b_ref[...], 0.0)

    @pl.when(i == 0)
    def _():
        colsum[...] = jnp.zeros_like(colsum)

    colsum[...] += jnp.sum(h2, axis=0, keepdims=True)

    @pl.when(i == pl.num_programs(0) - 1)
    def _():
        out_ref[...] = (
            jnp.dot(colsum[...] * (1.0 / _N), wp_ref[...],
                    preferred_element_type=jnp.float32, precision=jax.lax.Precision.HIGHEST) + bp_ref[...])


def _tc_layer1(h, agg, deg2d, Ws, Wn, b):
    return pl.pallas_call(
        _layer1_body,
        grid=(_N // _R,),
        in_specs=[
            pl.BlockSpec((_R, _D), lambda i: (i, 0)),
            pl.BlockSpec((_NC, _R, _D), lambda i: (0, i, 0)),
            pl.BlockSpec((_R, 1), lambda i: (i, 0)),
            pl.BlockSpec((_D, _D), lambda i: (0, 0)),
            pl.BlockSpec((_D, _D), lambda i: (0, 0)),
            pl.BlockSpec((1, _D), lambda i: (0, 0)),
        ],
        out_specs=pl.BlockSpec((_R, _D), lambda i: (i, 0)),
        out_shape=jax.ShapeDtypeStruct((_N, _D), jnp.float32),
    )(h, agg, deg2d, Ws, Wn, b)


def _tc_layer2(h, agg, deg2d, Ws, Wn, b, Wp, bp):
    return pl.pallas_call(
        _layer2_body,
        grid=(_N // _R,),
        in_specs=[
            pl.BlockSpec((_R, _D), lambda i: (i, 0)),
            pl.BlockSpec((_NC, _R, _D), lambda i: (0, i, 0)),
            pl.BlockSpec((_R, 1), lambda i: (i, 0)),
            pl.BlockSpec((_D, _D), lambda i: (0, 0)),
            pl.BlockSpec((_D, _D), lambda i: (0, 0)),
            pl.BlockSpec((1, _D), lambda i: (0, 0)),
            pl.BlockSpec((_D, 1), lambda i: (0, 0)),
            pl.BlockSpec((1, 1), lambda i: (0, 0)),
        ],
        out_specs=pl.BlockSpec((1, 1), lambda i: (0, 0)),
        out_shape=jax.ShapeDtypeStruct((1, 1), jnp.float32),
        scratch_shapes=[pltpu.VMEM((1, _D), jnp.float32)],
    )(h, agg, deg2d, Ws, Wn, b, Wp, bp)


def kernel(x, edge_index, W_self1, W_neigh1, b1, W_self2, W_neigh2, b2, Wp, bp):
    npad = _EP - _E
    pad_iota = jnp.arange(npad, dtype=jnp.int32)
    src = jnp.concatenate(
        [edge_index[0].astype(jnp.int32), pad_iota % _N]
    ).reshape(_NW, _NSTG, _CPS, _CH)
    dst = jnp.concatenate(
        [edge_index[1].astype(jnp.int32), _N + pad_iota % _NPAD]
    ).reshape(_NW, _NSTG, _CPS, _CH)

    agg1, degp = _make_sc_agg(True)(x, src, dst)
    deg2d = (degp[0, :_N] + degp[1, :_N]).reshape(_N, 1)
    h1 = _tc_layer1(x, agg1, deg2d, W_self1, W_neigh1, b1.reshape(1, _D))
    agg2 = _make_sc_agg(False)(h1, src, dst)
    return _tc_layer2(h1, agg2, deg2d, W_self2, W_neigh2, b2.reshape(1, _D),
                      Wp, bp.reshape(1, 1))


# speedup vs baseline: 10.6460x; 1.0006x over previous
import functools

import jax
import jax.numpy as jnp
from jax import lax
from jax.experimental import pallas as pl
from jax.experimental.pallas import tpu as pltpu
from jax.experimental.pallas import tpu_sc as plsc

_N = 10000
_E = 320000
_D = 128

_NC = 2
_NS = 16
_NW = _NC * _NS
_CH = 128
_EP = 327680
_NSTG = 10
_CPS = 8
_NPAD = 240
_NA = _N + _NPAD
_RPS = 624
_TAIL = _N - _RPS * _NS
_DEG_PAD = 10240
_DPS = _DEG_PAD // _NS


def _sc_agg_body(with_deg, h_hbm, src_hbm, dst_hbm, *refs):
    if with_deg:
        (out_hbm, deg_hbm, srcv, dstv, rows_a, rows_b, onesv, acc, dacc,
         sem_a, sem_b, sem_i) = refs
    else:
        (out_hbm, srcv, dstv, rows_a, rows_b, onesv, acc, sem_a, sem_b,
         sem_i) = refs
    c = lax.axis_index("c")
    s = lax.axis_index("s")
    w = s * _NC + c

    zero16 = jnp.zeros((16,), jnp.float32)
    def _zrow(i, carry):
        rows_a[i // (_D // 16), pl.ds((i % (_D // 16)) * 16, 16)] = zero16
        return carry
    lax.fori_loop(0, _CH * _D // 16, _zrow, 0)
    one16 = jnp.full((16,), 1.0, jnp.float32)
    for i in range(_CH // 16):
        onesv[pl.ds(i * 16, 16)] = one16

    for t in range(_RPS // _CH):
        pltpu.sync_copy(rows_a, acc.at[pl.ds(s * _RPS + t * _CH, _CH)])
    rem = _RPS % _CH
    if rem:
        pltpu.sync_copy(rows_a.at[pl.ds(0, rem)],
                        acc.at[pl.ds(s * _RPS + (_RPS // _CH) * _CH, rem)])

    @pl.when(s == _NS - 1)
    def _():
        pltpu.sync_copy(rows_a.at[pl.ds(0, _TAIL)],
                        acc.at[pl.ds(_RPS * _NS, _TAIL)])

    if with_deg:
        for t in range(_DPS // _D):
            pltpu.sync_copy(rows_a.at[0], dacc.at[pl.ds(s * _DPS + t * _D, _D)])
    plsc.subcore_barrier()

    def _sidx(j):
        return srcv.at[(j // _CPS) % 2, j % _CPS]

    def _didx(j):
        return dstv.at[(j // _CPS) % 2, j % _CPS]

    def _g(j, buf):
        pltpu.async_copy(h_hbm.at[_sidx(j)], buf, sem_a)

    def _gw(j, buf):
        pltpu.make_async_copy(h_hbm.at[_sidx(j)], buf, sem_a).wait()

    def _s(j, buf):
        pltpu.async_copy(buf, acc.at[_didx(j)], sem_b, add=True)
        if with_deg:
            pltpu.sync_copy(onesv, dacc.at[_didx(j)], add=True)

    def _sw(j, buf):
        pltpu.make_async_copy(buf, acc.at[_didx(j)], sem_b).wait()

    def _ipf(st):
        pltpu.async_copy(src_hbm.at[w, st], srcv.at[st % 2], sem_i)
        pltpu.async_copy(dst_hbm.at[w, st], dstv.at[st % 2], sem_i)

    def _ipw(st):
        pltpu.make_async_copy(src_hbm.at[w, st], srcv.at[st % 2],
                              sem_i).wait()
        pltpu.make_async_copy(dst_hbm.at[w, st], dstv.at[st % 2],
                              sem_i).wait()

    pltpu.sync_copy(src_hbm.at[w, 0], srcv.at[0])
    pltpu.sync_copy(dst_hbm.at[w, 0], dstv.at[0])
    _ipf(1)
    _g(0, rows_a)
    _gw(0, rows_a)
    _g(1, rows_b)
    _s(0, rows_a)

    _nchunk = _NSTG * _CPS
    def _pair(k, carry):
        j0 = 2 * k
        st = j0 // _CPS
        at_entry = (j0 % _CPS) == 0

        @pl.when(at_entry)
        def _():
            _ipw(st)
        _gw(j0 - 1, rows_b)
        _sw(j0 - 2, rows_a)
        _g(j0, rows_a)
        _s(j0 - 1, rows_b)
        _gw(j0, rows_a)
        _sw(j0 - 1, rows_b)

        @pl.when(at_entry & (st < _NSTG - 1))
        def _():
            _ipf(st + 1)
        _g(j0 + 1, rows_b)
        _s(j0, rows_a)
        return carry
    lax.fori_loop(1, _nchunk // 2, _pair, 0)
    _gw(_nchunk - 1, rows_b)
    _sw(_nchunk - 2, rows_a)
    _s(_nchunk - 1, rows_b)
    _sw(_nchunk - 1, rows_b)
    plsc.subcore_barrier()

    pltpu.sync_copy(acc.at[pl.ds(s * _RPS, _RPS)],
                    out_hbm.at[c, pl.ds(s * _RPS, _RPS)])

    @pl.when(s == _NS - 1)
    def _():
        pltpu.sync_copy(acc.at[pl.ds(_RPS * _NS, _TAIL)],
                        out_hbm.at[c, pl.ds(_RPS * _NS, _TAIL)])

    if with_deg:
        pltpu.sync_copy(dacc.at[pl.ds(s * _DPS, _DPS)],
                        deg_hbm.at[c, pl.ds(s * _DPS, _DPS)])


def _make_sc_agg(with_deg):
    mesh = plsc.VectorSubcoreMesh(core_axis_name="c", subcore_axis_name="s")
    out_type = [jax.ShapeDtypeStruct((_NC, _N, _D), jnp.float32)]
    if with_deg:
        out_type.append(jax.ShapeDtypeStruct((_NC, _DEG_PAD), jnp.float32))
    scratch = [
        pltpu.VMEM((2, _CPS, _CH), jnp.int32),
        pltpu.VMEM((2, _CPS, _CH), jnp.int32),
        pltpu.VMEM((_CH, _D), jnp.float32),
        pltpu.VMEM((_CH, _D), jnp.float32),
        pltpu.VMEM((_CH,), jnp.float32),
        pltpu.VMEM_SHARED((_NA, _D), jnp.float32),
    ]
    if with_deg:
        scratch.append(pltpu.VMEM_SHARED((_DEG_PAD,), jnp.float32))
    scratch.append(pltpu.SemaphoreType.DMA)
    scratch.append(pltpu.SemaphoreType.DMA)
    scratch.append(pltpu.SemaphoreType.DMA)
    return pl.kernel(
        functools.partial(_sc_agg_body, with_deg),
        out_type=tuple(out_type) if with_deg else out_type[0],
        mesh=mesh,
        scratch_types=scratch,
    )


_R = 2000


def _layer1_body(h_ref, agg_ref, deg_ref, ws_ref, wn_ref, b_ref, out_ref):
    a = agg_ref[0] + agg_ref[1]
    inv = 1.0 / jnp.maximum(deg_ref[...], 1.0)
    out_ref[...] = jnp.maximum(
        jnp.dot(h_ref[...], ws_ref[...], preferred_element_type=jnp.float32, precision=jax.lax.Precision.HIGHEST)
        + jnp.dot(a * inv, wn_ref[...], preferred_element_type=jnp.float32, precision=jax.lax.Precision.HIGHEST)
        + b_ref[...], 0.0)


def _layer2_body(h_ref, agg_ref, deg_ref, ws_ref, wn_ref, b_ref, wp_ref,
                 bp_ref, out_ref, colsum):
    i = pl.program_id(0)
    a = agg_ref[0] + agg_ref[1]
    inv = 1.0 / jnp.maximum(deg_ref[...], 1.0)
    h2 = jnp.maximum(
        jnp.dot(h_ref[...], ws_ref[...], preferred_element_type=jnp.float32, precision=jax.lax.Precision.HIGHEST)
        + jnp.dot(a * inv, wn_ref[...], preferred_element_type=jnp.float32, precision=jax.lax.Precision.HIGHEST)
        + b_ref[...], 0.0)

    @pl.when(i == 0)
    def _():
        colsum[...] = jnp.zeros_like(colsum)

    colsum[...] += jnp.sum(h2, axis=0, keepdims=True)

    @pl.when(i == pl.num_programs(0) - 1)
    def _():
        out_ref[...] = (
            jnp.dot(colsum[...] * (1.0 / _N), wp_ref[...],
                    preferred_element_type=jnp.float32, precision=jax.lax.Precision.HIGHEST) + bp_ref[...])


def _tc_layer1(h, agg, deg2d, Ws, Wn, b):
    return pl.pallas_call(
        _layer1_body,
        grid=(_N // _R,),
        in_specs=[
            pl.BlockSpec((_R, _D), lambda i: (i, 0)),
            pl.BlockSpec((_NC, _R, _D), lambda i: (0, i, 0)),
            pl.BlockSpec((_R, 1), lambda i: (i, 0)),
            pl.BlockSpec((_D, _D), lambda i: (0, 0)),
            pl.BlockSpec((_D, _D), lambda i: (0, 0)),
            pl.BlockSpec((1, _D), lambda i: (0, 0)),
        ],
        out_specs=pl.BlockSpec((_R, _D), lambda i: (i, 0)),
        out_shape=jax.ShapeDtypeStruct((_N, _D), jnp.float32),
    )(h, agg, deg2d, Ws, Wn, b)


def _tc_layer2(h, agg, deg2d, Ws, Wn, b, Wp, bp):
    return pl.pallas_call(
        _layer2_body,
        grid=(_N // _R,),
        in_specs=[
            pl.BlockSpec((_R, _D), lambda i: (i, 0)),
            pl.BlockSpec((_NC, _R, _D), lambda i: (0, i, 0)),
            pl.BlockSpec((_R, 1), lambda i: (i, 0)),
            pl.BlockSpec((_D, _D), lambda i: (0, 0)),
            pl.BlockSpec((_D, _D), lambda i: (0, 0)),
            pl.BlockSpec((1, _D), lambda i: (0, 0)),
            pl.BlockSpec((_D, 1), lambda i: (0, 0)),
            pl.BlockSpec((1, 1), lambda i: (0, 0)),
        ],
        out_specs=pl.BlockSpec((1, 1), lambda i: (0, 0)),
        out_shape=jax.ShapeDtypeStruct((1, 1), jnp.float32),
        scratch_shapes=[pltpu.VMEM((1, _D), jnp.float32)],
    )(h, agg, deg2d, Ws, Wn, b, Wp, bp)


def kernel(x, edge_index, W_self1, W_neigh1, b1, W_self2, W_neigh2, b2, Wp, bp):
    npad = _EP - _E
    pad_iota = jnp.arange(npad, dtype=jnp.int32)
    src = jnp.concatenate(
        [edge_index[0].astype(jnp.int32), pad_iota % _N]
    ).reshape(_NW, _NSTG, _CPS, _CH)
    dst = jnp.concatenate(
        [edge_index[1].astype(jnp.int32), _N + pad_iota % _NPAD]
    ).reshape(_NW, _NSTG, _CPS, _CH)

    agg1, degp = _make_sc_agg(True)(x, src, dst)
    deg2d = (degp[0, :_N] + degp[1, :_N]).reshape(_N, 1)
    h1 = _tc_layer1(x, agg1, deg2d, W_self1, W_neigh1, b1.reshape(1, _D))
    agg2 = _make_sc_agg(False)(h1, src, dst)
    return _tc_layer2(h1, agg2, deg2d, W_self2, W_neigh2, b2.reshape(1, _D),
                      Wp, bp.reshape(1, 1))
